# initial kernel scaffold (unmeasured)
import jax
import jax.numpy as jnp
from jax import lax
from jax.experimental import pallas as pl
from jax.experimental.pallas import tpu as pltpu

N_LAYERS = 3


def kernel(x, Win0, Wout0, Win1, Wout1, Win2, Wout2):
    b, d_in = x.shape
    k_win, h_blk = Win0.shape
    k_wout, d_out = Wout0.shape

    def body(
        x_ref, win0_ref, wout0_ref, win1_ref, wout1_ref, win2_ref, wout2_ref,
        out_ref,
        win_buf, wout_buf,
        send_y, recv_y, send_x, recv_x,
        w_sems, y_send_sems, y_recv_sems, x_send_sems, x_recv_sems,
    ):
        my_x = lax.axis_index("x")
        my_y = lax.axis_index("y")
        partner_y = (my_x, 1 - my_y)
        partner_x = (1 - my_x, my_y)

        win_refs = [win0_ref, win1_ref, win2_ref]
        wout_refs = [wout0_ref, wout1_ref, wout2_ref]
        win_cp = [
            pltpu.make_async_copy(win_refs[l], win_buf, w_sems.at[2 * l])
            for l in range(N_LAYERS)
        ]
        wout_cp = [
            pltpu.make_async_copy(wout_refs[l], wout_buf, w_sems.at[2 * l + 1])
            for l in range(N_LAYERS)
        ]

        win_cp[0].start()
        wout_cp[0].start()

        barrier = pltpu.get_barrier_semaphore()
        for nbr in (partner_y, partner_x):
            pl.semaphore_signal(
                barrier, inc=1, device_id=nbr,
                device_id_type=pl.DeviceIdType.MESH,
            )
        pl.semaphore_wait(barrier, 2)

        act = x_ref[...].astype(jnp.bfloat16)

        for l in range(N_LAYERS):
            win_cp[l].wait()
            p = jnp.dot(
                act, win_buf[...].astype(jnp.bfloat16),
                preferred_element_type=jnp.float32,
            )
            send_y[l, :, :] = p.astype(jnp.bfloat16)
            rdma_y = pltpu.make_async_remote_copy(
                src_ref=send_y.at[l], dst_ref=recv_y.at[l],
                send_sem=y_send_sems.at[l], recv_sem=y_recv_sems.at[l],
                device_id=partner_y, device_id_type=pl.DeviceIdType.MESH,
            )
            rdma_y.start()
            if l + 1 < N_LAYERS:
                win_cp[l + 1].start()
            rdma_y.wait()
            h = jnp.maximum(
                send_y[l].astype(jnp.float32) + recv_y[l].astype(jnp.float32),
                0.0,
            ).astype(jnp.bfloat16)

            wout_cp[l].wait()
            q = jnp.dot(
                h, wout_buf[...].astype(jnp.bfloat16),
                preferred_element_type=jnp.float32,
            )
            send_x[l, :, :] = q.astype(jnp.bfloat16)
            rdma_x = pltpu.make_async_remote_copy(
                src_ref=send_x.at[l], dst_ref=recv_x.at[l],
                send_sem=x_send_sems.at[l], recv_sem=x_recv_sems.at[l],
                device_id=partner_x, device_id_type=pl.DeviceIdType.MESH,
            )
            rdma_x.start()
            if l + 1 < N_LAYERS:
                wout_cp[l + 1].start()
            rdma_x.wait()
            s = send_x[l].astype(jnp.float32) + recv_x[l].astype(jnp.float32)
            if l + 1 < N_LAYERS:
                act = s.astype(jnp.bfloat16)
            else:
                out_ref[...] = s

    return pl.pallas_call(
        body,
        out_shape=jax.ShapeDtypeStruct((b, d_out), jnp.float32),
        in_specs=[
            pl.BlockSpec(memory_space=pltpu.VMEM),
            pl.BlockSpec(memory_space=pltpu.ANY),
            pl.BlockSpec(memory_space=pltpu.ANY),
            pl.BlockSpec(memory_space=pltpu.ANY),
            pl.BlockSpec(memory_space=pltpu.ANY),
            pl.BlockSpec(memory_space=pltpu.ANY),
            pl.BlockSpec(memory_space=pltpu.ANY),
        ],
        out_specs=pl.BlockSpec(memory_space=pltpu.VMEM),
        scratch_shapes=[
            pltpu.VMEM((k_win, h_blk), jnp.float32),
            pltpu.VMEM((k_wout, d_out), jnp.float32),
            pltpu.VMEM((N_LAYERS, b, h_blk), jnp.bfloat16),
            pltpu.VMEM((N_LAYERS, b, h_blk), jnp.bfloat16),
            pltpu.VMEM((N_LAYERS, b, d_out), jnp.bfloat16),
            pltpu.VMEM((N_LAYERS, b, d_out), jnp.bfloat16),
            pltpu.SemaphoreType.DMA((2 * N_LAYERS,)),
            pltpu.SemaphoreType.DMA((N_LAYERS,)),
            pltpu.SemaphoreType.DMA((N_LAYERS,)),
            pltpu.SemaphoreType.DMA((N_LAYERS,)),
            pltpu.SemaphoreType.DMA((N_LAYERS,)),
        ],
        compiler_params=pltpu.CompilerParams(collective_id=0),
    )(x, Win0, Wout0, Win1, Wout1, Win2, Wout2)


# baseline (device time: 128188 ns/iter reference)
import jax
import jax.numpy as jnp
from jax import lax
from jax.experimental import pallas as pl
from jax.experimental.pallas import tpu as pltpu

N_LAYERS = 3
CHUNK = 512


def kernel(x, Win0, Wout0, Win1, Wout1, Win2, Wout2):
    b, d_in = x.shape
    k_win, h_blk = Win0.shape
    k_wout, d_out = Wout0.shape
    n_win_chunks = k_win // CHUNK
    n_wout_chunks = k_wout // CHUNK

    def body(
        x_ref, win0_ref, wout0_ref, win1_ref, wout1_ref, win2_ref, wout2_ref,
        out_ref,
        win_buf, wout_buf,
        send_y, recv_y, send_x, recv_x,
        win_sems, wout_sems, y_send_sems, y_recv_sems, x_send_sems, x_recv_sems,
    ):
        my_x = lax.axis_index("x")
        my_y = lax.axis_index("y")
        partner_y = (my_x, 1 - my_y)
        partner_x = (1 - my_x, my_y)

        def chunk_copies(w_ref, buf, sems, n_chunks):
            return [
                pltpu.make_async_copy(
                    w_ref.at[pl.ds(c * CHUNK, CHUNK), :],
                    buf.at[c % 2],
                    sems.at[c % 2],
                )
                for c in range(n_chunks)
            ]

        win_cps = [
            chunk_copies(w, win_buf, win_sems, n_win_chunks)
            for w in (win0_ref, win1_ref, win2_ref)
        ]
        wout_cps = [
            chunk_copies(w, wout_buf, wout_sems, n_wout_chunks)
            for w in (wout0_ref, wout1_ref, wout2_ref)
        ]

        def stream_matmul(a, cps, buf, n_chunks):
            acc = None
            for c in range(n_chunks):
                cps[c].wait()
                if c + 1 < n_chunks:
                    cps[c + 1].start()
                part = jnp.dot(
                    a[:, c * CHUNK:(c + 1) * CHUNK],
                    buf[c % 2].astype(jnp.bfloat16),
                    preferred_element_type=jnp.float32,
                )
                acc = part if acc is None else acc + part
            return acc

        win_cps[0][0].start()

        barrier = pltpu.get_barrier_semaphore()
        for nbr in (partner_y, partner_x):
            pl.semaphore_signal(
                barrier, inc=1, device_id=nbr,
                device_id_type=pl.DeviceIdType.MESH,
            )
        pl.semaphore_wait(barrier, 2)

        act = x_ref[...].astype(jnp.bfloat16)

        for l in range(N_LAYERS):
            p = stream_matmul(act, win_cps[l], win_buf, n_win_chunks)
            send_y[l, :, :] = p.astype(jnp.bfloat16)
            rdma_y = pltpu.make_async_remote_copy(
                src_ref=send_y.at[l], dst_ref=recv_y.at[l],
                send_sem=y_send_sems.at[l], recv_sem=y_recv_sems.at[l],
                device_id=partner_y, device_id_type=pl.DeviceIdType.MESH,
            )
            rdma_y.start()
            wout_cps[l][0].start()
            rdma_y.wait()
            h = jnp.maximum(
                send_y[l].astype(jnp.float32) + recv_y[l].astype(jnp.float32),
                0.0,
            ).astype(jnp.bfloat16)

            q = stream_matmul(h, wout_cps[l], wout_buf, n_wout_chunks)
            send_x[l, :, :] = q.astype(jnp.bfloat16)
            rdma_x = pltpu.make_async_remote_copy(
                src_ref=send_x.at[l], dst_ref=recv_x.at[l],
                send_sem=x_send_sems.at[l], recv_sem=x_recv_sems.at[l],
                device_id=partner_x, device_id_type=pl.DeviceIdType.MESH,
            )
            rdma_x.start()
            if l + 1 < N_LAYERS:
                win_cps[l + 1][0].start()
            rdma_x.wait()
            s = send_x[l].astype(jnp.float32) + recv_x[l].astype(jnp.float32)
            if l + 1 < N_LAYERS:
                act = s.astype(jnp.bfloat16)
            else:
                out_ref[...] = s

    return pl.pallas_call(
        body,
        out_shape=jax.ShapeDtypeStruct((b, d_out), jnp.float32),
        in_specs=[
            pl.BlockSpec(memory_space=pltpu.VMEM),
            pl.BlockSpec(memory_space=pl.ANY),
            pl.BlockSpec(memory_space=pl.ANY),
            pl.BlockSpec(memory_space=pl.ANY),
            pl.BlockSpec(memory_space=pl.ANY),
            pl.BlockSpec(memory_space=pl.ANY),
            pl.BlockSpec(memory_space=pl.ANY),
        ],
        out_specs=pl.BlockSpec(memory_space=pltpu.VMEM),
        scratch_shapes=[
            pltpu.VMEM((2, CHUNK, h_blk), jnp.float32),
            pltpu.VMEM((2, CHUNK, d_out), jnp.float32),
            pltpu.VMEM((N_LAYERS, b, h_blk), jnp.bfloat16),
            pltpu.VMEM((N_LAYERS, b, h_blk), jnp.bfloat16),
            pltpu.VMEM((N_LAYERS, b, d_out), jnp.bfloat16),
            pltpu.VMEM((N_LAYERS, b, d_out), jnp.bfloat16),
            pltpu.SemaphoreType.DMA((2,)),
            pltpu.SemaphoreType.DMA((2,)),
            pltpu.SemaphoreType.DMA((N_LAYERS,)),
            pltpu.SemaphoreType.DMA((N_LAYERS,)),
            pltpu.SemaphoreType.DMA((N_LAYERS,)),
            pltpu.SemaphoreType.DMA((N_LAYERS,)),
        ],
        compiler_params=pltpu.CompilerParams(
            collective_id=0,
            vmem_limit_bytes=60 * 1024 * 1024,
        ),
    )(x, Win0, Wout0, Win1, Wout1, Win2, Wout2)


# device time: 90835 ns/iter; 1.4112x vs baseline; 1.4112x over previous
import jax
import jax.numpy as jnp
from jax import lax
from jax.experimental import pallas as pl
from jax.experimental.pallas import tpu as pltpu

N_LAYERS = 3
CHUNK = 512
WIN_DEPTH = 3
WOUT_DEPTH = 4


def kernel(x, Win0, Wout0, Win1, Wout1, Win2, Wout2):
    b, d_in = x.shape
    k_win, h_blk = Win0.shape
    k_wout, d_out = Wout0.shape
    n_win_chunks = k_win // CHUNK
    n_wout_chunks = k_wout // CHUNK

    def body(
        x_ref, win0_ref, wout0_ref, win1_ref, wout1_ref, win2_ref, wout2_ref,
        out_ref,
        win_buf, wout_buf,
        send_y, recv_y, send_x, recv_x,
        win_sems, wout_sems, y_send_sems, y_recv_sems, x_send_sems, x_recv_sems,
    ):
        my_x = lax.axis_index("x")
        my_y = lax.axis_index("y")
        partner_y = (my_x, 1 - my_y)
        partner_x = (1 - my_x, my_y)

        def make_stream(w_refs, per_layer, buf, sems, depth):
            cps = []
            for w in w_refs:
                for c in range(per_layer):
                    g = len(cps)
                    cps.append(
                        pltpu.make_async_copy(
                            w.at[pl.ds(c * CHUNK, CHUNK), :],
                            buf.at[g % depth],
                            sems.at[g % depth],
                        )
                    )
            return {"cps": cps, "depth": depth, "started": 0, "consumed": 0}

        win_st = make_stream(
            (win0_ref, win1_ref, win2_ref), n_win_chunks,
            win_buf, win_sems, WIN_DEPTH,
        )
        wout_st = make_stream(
            (wout0_ref, wout1_ref, wout2_ref), n_wout_chunks,
            wout_buf, wout_sems, WOUT_DEPTH,
        )

        def topup(st):
            while (
                st["started"] < len(st["cps"])
                and st["started"] < st["consumed"] + st["depth"]
            ):
                st["cps"][st["started"]].start()
                st["started"] += 1

        def stream_matmul(a, st, buf, n_chunks):
            acc = None
            for c in range(n_chunks):
                topup(st)
                g = st["consumed"]
                st["cps"][g].wait()
                part = jnp.dot(
                    a[:, c * CHUNK:(c + 1) * CHUNK],
                    buf[g % st["depth"]].astype(jnp.bfloat16),
                    preferred_element_type=jnp.float32,
                )
                st["consumed"] += 1
                topup(st)
                acc = part if acc is None else acc + part
            return acc

        topup(win_st)

        barrier = pltpu.get_barrier_semaphore()
        for nbr in (partner_y, partner_x):
            pl.semaphore_signal(
                barrier, inc=1, device_id=nbr,
                device_id_type=pl.DeviceIdType.MESH,
            )
        pl.semaphore_wait(barrier, 2)

        act = x_ref[...].astype(jnp.bfloat16)

        for l in range(N_LAYERS):
            p = stream_matmul(act, win_st, win_buf, n_win_chunks)
            send_y[l, :, :] = p.astype(jnp.bfloat16)
            rdma_y = pltpu.make_async_remote_copy(
                src_ref=send_y.at[l], dst_ref=recv_y.at[l],
                send_sem=y_send_sems.at[l], recv_sem=y_recv_sems.at[l],
                device_id=partner_y, device_id_type=pl.DeviceIdType.MESH,
            )
            rdma_y.start()
            topup(wout_st)
            rdma_y.wait()
            h = jnp.maximum(
                send_y[l].astype(jnp.float32) + recv_y[l].astype(jnp.float32),
                0.0,
            ).astype(jnp.bfloat16)

            q = stream_matmul(h, wout_st, wout_buf, n_wout_chunks)
            send_x[l, :, :] = q.astype(jnp.bfloat16)
            rdma_x = pltpu.make_async_remote_copy(
                src_ref=send_x.at[l], dst_ref=recv_x.at[l],
                send_sem=x_send_sems.at[l], recv_sem=x_recv_sems.at[l],
                device_id=partner_x, device_id_type=pl.DeviceIdType.MESH,
            )
            rdma_x.start()
            topup(win_st)
            rdma_x.wait()
            s = send_x[l].astype(jnp.float32) + recv_x[l].astype(jnp.float32)
            if l + 1 < N_LAYERS:
                act = s.astype(jnp.bfloat16)
            else:
                out_ref[...] = s

    return pl.pallas_call(
        body,
        out_shape=jax.ShapeDtypeStruct((b, d_out), jnp.float32),
        in_specs=[
            pl.BlockSpec(memory_space=pltpu.VMEM),
            pl.BlockSpec(memory_space=pl.ANY),
            pl.BlockSpec(memory_space=pl.ANY),
            pl.BlockSpec(memory_space=pl.ANY),
            pl.BlockSpec(memory_space=pl.ANY),
            pl.BlockSpec(memory_space=pl.ANY),
            pl.BlockSpec(memory_space=pl.ANY),
        ],
        out_specs=pl.BlockSpec(memory_space=pltpu.VMEM),
        scratch_shapes=[
            pltpu.VMEM((WIN_DEPTH, CHUNK, h_blk), jnp.float32),
            pltpu.VMEM((WOUT_DEPTH, CHUNK, d_out), jnp.float32),
            pltpu.VMEM((N_LAYERS, b, h_blk), jnp.bfloat16),
            pltpu.VMEM((N_LAYERS, b, h_blk), jnp.bfloat16),
            pltpu.VMEM((N_LAYERS, b, d_out), jnp.bfloat16),
            pltpu.VMEM((N_LAYERS, b, d_out), jnp.bfloat16),
            pltpu.SemaphoreType.DMA((WIN_DEPTH,)),
            pltpu.SemaphoreType.DMA((WOUT_DEPTH,)),
            pltpu.SemaphoreType.DMA((N_LAYERS,)),
            pltpu.SemaphoreType.DMA((N_LAYERS,)),
            pltpu.SemaphoreType.DMA((N_LAYERS,)),
            pltpu.SemaphoreType.DMA((N_LAYERS,)),
        ],
        compiler_params=pltpu.CompilerParams(
            collective_id=0,
            vmem_limit_bytes=60 * 1024 * 1024,
        ),
    )(x, Win0, Wout0, Win1, Wout1, Win2, Wout2)


# device time: 79251 ns/iter; 1.6175x vs baseline; 1.1462x over previous
import jax
import jax.numpy as jnp
from jax import lax
from jax.experimental import pallas as pl
from jax.experimental.pallas import tpu as pltpu

N_LAYERS = 3
RCHUNK = 512
WIN_DEPTH = 6
WOUT_DEPTH = 8


def kernel(x, Win0, Wout0, Win1, Wout1, Win2, Wout2):
    b, d_in = x.shape
    k_win, h_blk = Win0.shape
    k_wout, d_out = Wout0.shape
    h_half = h_blk // 2
    d_half = d_out // 2
    n_win_rows = k_win // RCHUNK
    n_wout_rows = k_wout // RCHUNK

    def body(
        x_ref, win0_ref, wout0_ref, win1_ref, wout1_ref, win2_ref, wout2_ref,
        out_ref,
        win_buf, wout_buf,
        send_y, recv_y, send_x, recv_x,
        win_sems, wout_sems, y_send_sems, y_recv_sems, x_send_sems, x_recv_sems,
    ):
        my_x = lax.axis_index("x")
        my_y = lax.axis_index("y")
        partner_y = (my_x, 1 - my_y)
        partner_x = (1 - my_x, my_y)

        def make_stream(w_refs, n_rows, col_w, buf, sems, depth):
            cps = []
            for w in w_refs:
                for col in range(2):
                    for r in range(n_rows):
                        g = len(cps)
                        cps.append(
                            pltpu.make_async_copy(
                                w.at[
                                    pl.ds(r * RCHUNK, RCHUNK),
                                    pl.ds(col * col_w, col_w),
                                ],
                                buf.at[g % depth],
                                sems.at[g % depth],
                            )
                        )
            return {"cps": cps, "depth": depth, "started": 0, "consumed": 0}

        win_st = make_stream(
            (win0_ref, win1_ref, win2_ref), n_win_rows, h_half,
            win_buf, win_sems, WIN_DEPTH,
        )
        wout_st = make_stream(
            (wout0_ref, wout1_ref, wout2_ref), n_wout_rows, d_half,
            wout_buf, wout_sems, WOUT_DEPTH,
        )

        def topup(st):
            while (
                st["started"] < len(st["cps"])
                and st["started"] < st["consumed"] + st["depth"]
            ):
                st["cps"][st["started"]].start()
                st["started"] += 1

        def consume(st, buf, a_pieces, acc):
            for ap in a_pieces:
                topup(st)
                g = st["consumed"]
                st["cps"][g].wait()
                part = jnp.dot(
                    ap,
                    buf[g % st["depth"]].astype(jnp.bfloat16),
                    preferred_element_type=jnp.float32,
                )
                st["consumed"] += 1
                topup(st)
                acc = part if acc is None else acc + part
            return acc

        def exchange(send_buf, recv_buf, slot, val, send_sems, recv_sems, tgt):
            send_buf[slot, :, :] = val.astype(jnp.bfloat16)
            rdma = pltpu.make_async_remote_copy(
                src_ref=send_buf.at[slot], dst_ref=recv_buf.at[slot],
                send_sem=send_sems.at[slot], recv_sem=recv_sems.at[slot],
                device_id=tgt, device_id_type=pl.DeviceIdType.MESH,
            )
            rdma.start()
            return rdma

        def combine(send_buf, recv_buf, slot):
            return (
                send_buf[slot].astype(jnp.float32)
                + recv_buf[slot].astype(jnp.float32)
            )

        topup(win_st)

        barrier = pltpu.get_barrier_semaphore()
        for nbr in (partner_y, partner_x):
            pl.semaphore_signal(
                barrier, inc=1, device_id=nbr,
                device_id_type=pl.DeviceIdType.MESH,
            )
        pl.semaphore_wait(barrier, 2)

        act = x_ref[...].astype(jnp.bfloat16)

        for l in range(N_LAYERS):
            a_pieces = [
                act[:, r * RCHUNK:(r + 1) * RCHUNK] for r in range(n_win_rows)
            ]
            p0 = consume(win_st, win_buf, a_pieces, None)
            ry0 = exchange(send_y, recv_y, 2 * l, p0,
                           y_send_sems, y_recv_sems, partner_y)
            p1 = consume(win_st, win_buf, a_pieces, None)
            ry1 = exchange(send_y, recv_y, 2 * l + 1, p1,
                           y_send_sems, y_recv_sems, partner_y)
            topup(wout_st)

            ry0.wait()
            h0 = jnp.maximum(combine(send_y, recv_y, 2 * l), 0.0).astype(
                jnp.bfloat16
            )
            h0_pieces = [
                h0[:, r * RCHUNK:(r + 1) * RCHUNK]
                for r in range(n_wout_rows // 2)
            ]
            q0 = consume(wout_st, wout_buf, h0_pieces, None)
            ry1.wait()
            h1 = jnp.maximum(combine(send_y, recv_y, 2 * l + 1), 0.0).astype(
                jnp.bfloat16
            )
            h1_pieces = [
                h1[:, r * RCHUNK:(r + 1) * RCHUNK]
                for r in range(n_wout_rows // 2)
            ]
            q0 = consume(wout_st, wout_buf, h1_pieces, q0)
            rx0 = exchange(send_x, recv_x, 2 * l, q0,
                           x_send_sems, x_recv_sems, partner_x)
            q1 = consume(wout_st, wout_buf, h0_pieces, None)
            q1 = consume(wout_st, wout_buf, h1_pieces, q1)
            rx1 = exchange(send_x, recv_x, 2 * l + 1, q1,
                           x_send_sems, x_recv_sems, partner_x)
            topup(win_st)

            rx0.wait()
            s0 = combine(send_x, recv_x, 2 * l)
            rx1.wait()
            s1 = combine(send_x, recv_x, 2 * l + 1)
            if l + 1 < N_LAYERS:
                act = jnp.concatenate([s0, s1], axis=1).astype(jnp.bfloat16)
            else:
                out_ref[:, :d_half] = s0
                out_ref[:, d_half:] = s1

    return pl.pallas_call(
        body,
        out_shape=jax.ShapeDtypeStruct((b, d_out), jnp.float32),
        in_specs=[
            pl.BlockSpec(memory_space=pltpu.VMEM),
            pl.BlockSpec(memory_space=pl.ANY),
            pl.BlockSpec(memory_space=pl.ANY),
            pl.BlockSpec(memory_space=pl.ANY),
            pl.BlockSpec(memory_space=pl.ANY),
            pl.BlockSpec(memory_space=pl.ANY),
            pl.BlockSpec(memory_space=pl.ANY),
        ],
        out_specs=pl.BlockSpec(memory_space=pltpu.VMEM),
        scratch_shapes=[
            pltpu.VMEM((WIN_DEPTH, RCHUNK, h_half), jnp.float32),
            pltpu.VMEM((WOUT_DEPTH, RCHUNK, d_half), jnp.float32),
            pltpu.VMEM((2 * N_LAYERS, b, h_half), jnp.bfloat16),
            pltpu.VMEM((2 * N_LAYERS, b, h_half), jnp.bfloat16),
            pltpu.VMEM((2 * N_LAYERS, b, d_half), jnp.bfloat16),
            pltpu.VMEM((2 * N_LAYERS, b, d_half), jnp.bfloat16),
            pltpu.SemaphoreType.DMA((WIN_DEPTH,)),
            pltpu.SemaphoreType.DMA((WOUT_DEPTH,)),
            pltpu.SemaphoreType.DMA((2 * N_LAYERS,)),
            pltpu.SemaphoreType.DMA((2 * N_LAYERS,)),
            pltpu.SemaphoreType.DMA((2 * N_LAYERS,)),
            pltpu.SemaphoreType.DMA((2 * N_LAYERS,)),
        ],
        compiler_params=pltpu.CompilerParams(
            collective_id=0,
            vmem_limit_bytes=60 * 1024 * 1024,
        ),
    )(x, Win0, Wout0, Win1, Wout1, Win2, Wout2)


# device time: 77652 ns/iter; 1.6508x vs baseline; 1.0206x over previous
import os

import jax
import jax.numpy as jnp
from jax import lax
from jax.experimental import pallas as pl
from jax.experimental.pallas import tpu as pltpu

_ABLATE = os.environ.get("ABLATE", "")

N_LAYERS = 3
RCHUNK = 512
WIN_DEPTH = 6
WOUT_DEPTH = 10


def kernel(x, Win0, Wout0, Win1, Wout1, Win2, Wout2):
    b, d_in = x.shape
    k_win, h_blk = Win0.shape
    k_wout, d_out = Wout0.shape
    h_half = h_blk // 2
    d_half = d_out // 2
    n_win_rows = k_win // RCHUNK
    n_wout_rows = k_wout // RCHUNK

    def body(
        x_ref, win0_ref, wout0_ref, win1_ref, wout1_ref, win2_ref, wout2_ref,
        out_ref,
        win_buf, wout_buf,
        send_y, recv_y, send_x, recv_x,
        win_sems, wout_sems, y_send_sems, y_recv_sems, x_send_sems, x_recv_sems,
    ):
        my_x = lax.axis_index("x")
        my_y = lax.axis_index("y")
        partner_y = (my_x, 1 - my_y)
        partner_x = (1 - my_x, my_y)

        def make_stream(w_refs, n_rows, col_w, buf, sems, depth):
            cps = []
            for w in w_refs:
                for col in range(2):
                    for r in range(n_rows):
                        g = len(cps)
                        cps.append(
                            pltpu.make_async_copy(
                                w.at[
                                    pl.ds(r * RCHUNK, RCHUNK),
                                    pl.ds(col * col_w, col_w),
                                ],
                                buf.at[g % depth],
                                sems.at[g % depth],
                            )
                        )
            return {"cps": cps, "depth": depth, "started": 0, "consumed": 0}

        win_st = make_stream(
            (win0_ref, win1_ref, win2_ref), n_win_rows, h_half,
            win_buf, win_sems, WIN_DEPTH,
        )
        wout_st = make_stream(
            (wout0_ref, wout1_ref, wout2_ref), n_wout_rows, d_half,
            wout_buf, wout_sems, WOUT_DEPTH,
        )

        def topup(st):
            while (
                st["started"] < len(st["cps"])
                and st["started"] < st["consumed"] + st["depth"]
            ):
                st["cps"][st["started"]].start()
                st["started"] += 1

        def consume(st, buf, a_pieces, acc):
            for ap in a_pieces:
                topup(st)
                g = st["consumed"]
                st["cps"][g].wait()
                part = jnp.dot(
                    ap(),
                    buf[g % st["depth"]].astype(jnp.bfloat16),
                    preferred_element_type=jnp.float32,
                )
                st["consumed"] += 1
                topup(st)
                acc = part if acc is None else acc + part
            return acc

        def exchange(send_buf, recv_buf, slot, val, send_sems, recv_sems, tgt):
            send_buf[slot, :, :] = val.astype(jnp.bfloat16)
            rdma = pltpu.make_async_remote_copy(
                src_ref=send_buf.at[slot], dst_ref=recv_buf.at[slot],
                send_sem=send_sems.at[slot], recv_sem=recv_sems.at[slot],
                device_id=tgt, device_id_type=pl.DeviceIdType.MESH,
            )
            if _ABLATE != "comm":
                rdma.start()
            return rdma

        def combine(send_buf, recv_buf, slot):
            return (
                send_buf[slot].astype(jnp.float32)
                + recv_buf[slot].astype(jnp.float32)
            )

        topup(win_st)

        barrier = pltpu.get_barrier_semaphore()
        for nbr in (partner_y, partner_x):
            pl.semaphore_signal(
                barrier, inc=1, device_id=nbr,
                device_id_type=pl.DeviceIdType.MESH,
            )
        pl.semaphore_wait(barrier, 2)

        act = x_ref[...].astype(jnp.bfloat16)
        rpd = d_half // RCHUNK

        def act_pieces_from(prev_rx):
            memo = {}

            def get_half(half):
                if half not in memo:
                    if _ABLATE != "comm":
                        prev_rx[half].wait()
                    memo[half] = combine(
                        send_x, recv_x, prev_rx["base"] + half
                    ).astype(jnp.bfloat16)
                return memo[half]

            return [
                (lambda r=r: get_half(r // rpd)[
                    :, (r % rpd) * RCHUNK:(r % rpd + 1) * RCHUNK
                ])
                for r in range(n_win_rows)
            ]

        a_pieces = [
            (lambda r=r: act[:, r * RCHUNK:(r + 1) * RCHUNK])
            for r in range(n_win_rows)
        ]

        for l in range(N_LAYERS):
            p0 = consume(win_st, win_buf, a_pieces, None)
            ry0 = exchange(send_y, recv_y, 2 * l, p0,
                           y_send_sems, y_recv_sems, partner_y)
            p1 = consume(win_st, win_buf, a_pieces, None)
            ry1 = exchange(send_y, recv_y, 2 * l + 1, p1,
                           y_send_sems, y_recv_sems, partner_y)
            topup(wout_st)

            if _ABLATE != "comm":
                ry0.wait()
            h0 = jnp.maximum(combine(send_y, recv_y, 2 * l), 0.0).astype(
                jnp.bfloat16
            )
            h0_pieces = [
                (lambda r=r: h0[:, r * RCHUNK:(r + 1) * RCHUNK])
                for r in range(n_wout_rows // 2)
            ]
            q0 = consume(wout_st, wout_buf, h0_pieces, None)
            if _ABLATE != "comm":
                ry1.wait()
            h1 = jnp.maximum(combine(send_y, recv_y, 2 * l + 1), 0.0).astype(
                jnp.bfloat16
            )
            h1_pieces = [
                (lambda r=r: h1[:, r * RCHUNK:(r + 1) * RCHUNK])
                for r in range(n_wout_rows // 2)
            ]
            q0 = consume(wout_st, wout_buf, h1_pieces, q0)
            rx0 = exchange(send_x, recv_x, 2 * l, q0,
                           x_send_sems, x_recv_sems, partner_x)
            q1 = consume(wout_st, wout_buf, h0_pieces, None)
            q1 = consume(wout_st, wout_buf, h1_pieces, q1)
            rx1 = exchange(send_x, recv_x, 2 * l + 1, q1,
                           x_send_sems, x_recv_sems, partner_x)
            topup(win_st)

            if l + 1 < N_LAYERS:
                a_pieces = act_pieces_from(
                    {0: rx0, 1: rx1, "base": 2 * l}
                )
            else:
                if _ABLATE != "comm":
                    rx0.wait()
                out_ref[:, :d_half] = combine(send_x, recv_x, 2 * l)
                if _ABLATE != "comm":
                    rx1.wait()
                out_ref[:, d_half:] = combine(send_x, recv_x, 2 * l + 1)

    return pl.pallas_call(
        body,
        out_shape=jax.ShapeDtypeStruct((b, d_out), jnp.float32),
        in_specs=[
            pl.BlockSpec(memory_space=pltpu.VMEM),
            pl.BlockSpec(memory_space=pl.ANY),
            pl.BlockSpec(memory_space=pl.ANY),
            pl.BlockSpec(memory_space=pl.ANY),
            pl.BlockSpec(memory_space=pl.ANY),
            pl.BlockSpec(memory_space=pl.ANY),
            pl.BlockSpec(memory_space=pl.ANY),
        ],
        out_specs=pl.BlockSpec(memory_space=pltpu.VMEM),
        scratch_shapes=[
            pltpu.VMEM((WIN_DEPTH, RCHUNK, h_half), jnp.float32),
            pltpu.VMEM((WOUT_DEPTH, RCHUNK, d_half), jnp.float32),
            pltpu.VMEM((2 * N_LAYERS, b, h_half), jnp.bfloat16),
            pltpu.VMEM((2 * N_LAYERS, b, h_half), jnp.bfloat16),
            pltpu.VMEM((2 * N_LAYERS, b, d_half), jnp.bfloat16),
            pltpu.VMEM((2 * N_LAYERS, b, d_half), jnp.bfloat16),
            pltpu.SemaphoreType.DMA((WIN_DEPTH,)),
            pltpu.SemaphoreType.DMA((WOUT_DEPTH,)),
            pltpu.SemaphoreType.DMA((2 * N_LAYERS,)),
            pltpu.SemaphoreType.DMA((2 * N_LAYERS,)),
            pltpu.SemaphoreType.DMA((2 * N_LAYERS,)),
            pltpu.SemaphoreType.DMA((2 * N_LAYERS,)),
        ],
        compiler_params=pltpu.CompilerParams(
            collective_id=0,
            vmem_limit_bytes=60 * 1024 * 1024,
        ),
    )(x, Win0, Wout0, Win1, Wout1, Win2, Wout2)
